# baseline (speedup 1.0000x reference)
import functools

import jax
import jax.numpy as jnp
from jax import lax
from jax.experimental import pallas as pl
from jax.experimental.pallas import tpu as pltpu
from jax.experimental.pallas import tpu_sc as plsc

GS = 128
F = 32
N = 2000000

NC = 2
NS = 16
NW = NC * NS

C = 640
NIDX = C // 128
NCHUNKS = N // C
L = 16

_RND = 8388608.0


def _tec_body(points_hbm, table_hbm, out_hbm, pts_v, idx_v, rows_v, sem):
    wid = lax.axis_index("s") * NC + lax.axis_index("c")
    nloc = (NCHUNKS - wid + NW - 1) // NW

    def body(i, carry):
        c = wid + i * NW
        base = c * C
        pltpu.sync_copy(points_hbm.at[pl.ds(base * 3, C * 3)], pts_v)
        lane = lax.iota(jnp.int32, L)
        for v in range(C // L):
            pi = (lane + v * L) * 3
            x = plsc.load_gather(pts_v, [pi])
            y = plsc.load_gather(pts_v, [pi + 1])
            z = plsc.load_gather(pts_v, [pi + 2])

            def rnd(t):
                t = jnp.clip(t * 127.0, 0.0, 127.0)
                return (t + _RND) - _RND

            flat = (rnd(x) * 16384.0 + rnd(y) * 128.0) + rnd(z)
            idx_v[v // 8, pl.ds((v % 8) * L, L)] = flat.astype(jnp.int32)
        copies = [
            pltpu.make_async_copy(
                table_hbm.at[idx_v.at[j]],
                rows_v.at[pl.ds(j * 128, 128)],
                sem,
            )
            for j in range(NIDX)
        ]
        for cp in copies:
            cp.start()
        for cp in copies:
            cp.wait()
        pltpu.sync_copy(rows_v, out_hbm.at[pl.ds(base, C)])
        return carry

    lax.fori_loop(0, nloc, body, 0)


@jax.jit
def _gather_features(points_flat, table):
    mesh = plsc.VectorSubcoreMesh(core_axis_name="c", subcore_axis_name="s")
    run = pl.kernel(
        _tec_body,
        out_type=jax.ShapeDtypeStruct((N, F), jnp.float32),
        mesh=mesh,
        compiler_params=pltpu.CompilerParams(
            needs_layout_passes=False, use_tc_tiling_on_sc=False
        ),
        scratch_types=[
            pltpu.VMEM((C * 3,), jnp.float32),
            pltpu.VMEM((NIDX, 128), jnp.int32),
            pltpu.VMEM((C, F), jnp.float32),
            pltpu.SemaphoreType.DMA,
        ],
    )
    return run(points_flat, table)


def kernel(points, grid):
    table = grid.reshape(GS * GS * GS, F)
    pts = points.reshape(-1)
    return _gather_features(pts, table)


# speedup vs baseline: 2.9327x; 2.9327x over previous
import jax
import jax.numpy as jnp
from jax import lax
from jax.experimental import pallas as pl
from jax.experimental.pallas import tpu as pltpu
from jax.experimental.pallas import tpu_sc as plsc
from jax.experimental.layout import Format, Layout

GS = 128
F = 32
N = 2000000
V = GS * GS * GS

NC = 2
NS = 16
NW = NC * NS

C = 640
NIDX = C // 128
NCHUNKS = N // C
NMAXH = (NCHUNKS + 2 * NW - 1) // (2 * NW)

YB = 32
PB = 3200
NPB = N // PB

_RND = 8388608.0


def _tr_body(g_ref, t_ref):
    for y in range(YB):
        t_ref[y * GS:(y + 1) * GS, :] = jnp.transpose(g_ref[0, y], (1, 0))


def _idx_body(p_ref, o_ref):
    def rnd(t):
        t = jnp.clip(t * (GS - 1.0), 0.0, GS - 1.0)
        return (t + _RND) - _RND

    x = rnd(p_ref[0:1, :])
    y = rnd(p_ref[1:2, :])
    z = rnd(p_ref[2:3, :])
    o_ref[0] = ((x * (GS * F) + y * F) * 4.0 + z).astype(jnp.int32)


def _sc_body(idx_hbm, table_hbm, out_hbm, idx_v, rows_v, si0, si1, sg0, sg1,
             so0, so1):
    wid = lax.axis_index("s") * NC + lax.axis_index("c")
    sem_in = (si0, si1)
    sem_g = (sg0, sg1)
    sem_out = (so0, so1)

    def in_copy(k, s):
        return pltpu.make_async_copy(
            idx_hbm.at[wid + k * NW], idx_v.at[s], sem_in[s]
        )

    def gather_copies(k, s):
        return [
            pltpu.make_async_copy(
                table_hbm.at[idx_v.at[s, j]],
                rows_v.at[s, pl.ds(j * 128, 128)],
                sem_g[s],
            )
            for j in range(NIDX)
        ]

    def out_copy(k, s):
        return pltpu.make_async_copy(
            out_hbm.at[pl.ds((wid + k * NW) * C, C)], rows_v.at[s], sem_out[s]
        )

    def valid(k):
        return wid + k * NW < NCHUNKS

    in_copy(0, 0).start()
    in_copy(1, 1).start()

    def outer(io, carry):
        for b in range(2):
            k = io * 2 + b

            @pl.when(valid(k))
            def _():
                in_copy(k, b).wait()

                @pl.when(io > 0)
                def _():
                    pltpu.make_async_copy(
                        out_hbm.at[pl.ds(0, C)], rows_v.at[b], sem_out[b]
                    ).wait()

                for cp in gather_copies(k, b):
                    cp.start()
                for cp in gather_copies(k, b):
                    cp.wait()

                @pl.when(valid(k + 2))
                def _():
                    in_copy(k + 2, b).start()

                cp = out_copy(k, b)
                cp.start()

        return carry

    lax.fori_loop(0, NMAXH, outer, 0)

    for b in range(2):
        pltpu.make_async_copy(
            out_hbm.at[pl.ds(0, C)], rows_v.at[b], sem_out[b]
        ).wait()


def _run(points, grid):
    g2 = jnp.transpose(grid, (0, 1, 3, 2))
    pts_t = jnp.transpose(points, (1, 0))

    table = pl.pallas_call(
        _tr_body,
        grid=(GS, GS // YB),
        in_specs=[
            pl.BlockSpec((1, YB, F, GS), lambda i, j: (i, j, 0, 0)),
        ],
        out_specs=pl.BlockSpec(
            (YB * GS, F), lambda i, j: (i * (GS // YB) + j, 0)
        ),
        out_shape=jax.ShapeDtypeStruct((V, F), jnp.float32),
    )(g2)

    idx = pl.pallas_call(
        _idx_body,
        grid=(NPB,),
        in_specs=[pl.BlockSpec((3, PB), lambda i: (0, i))],
        out_specs=pl.BlockSpec((1, 1, PB), lambda i: (i, 0, 0)),
        out_shape=jax.ShapeDtypeStruct((NPB, 1, PB), jnp.int32),
    )(pts_t)
    idx = idx.reshape(NCHUNKS, NIDX, 128)

    mesh = plsc.VectorSubcoreMesh(core_axis_name="c", subcore_axis_name="s")
    run = pl.kernel(
        _sc_body,
        out_type=jax.ShapeDtypeStruct((N, F), jnp.float32),
        mesh=mesh,
        compiler_params=pltpu.CompilerParams(
            needs_layout_passes=False, use_tc_tiling_on_sc=False
        ),
        scratch_types=[
            pltpu.VMEM((2, NIDX, 128), jnp.int32),
            pltpu.VMEM((2, C, F), jnp.float32),
            pltpu.SemaphoreType.DMA,
            pltpu.SemaphoreType.DMA,
            pltpu.SemaphoreType.DMA,
            pltpu.SemaphoreType.DMA,
            pltpu.SemaphoreType.DMA,
            pltpu.SemaphoreType.DMA,
        ],
    )
    return run(idx, table)


_jit_cache = {}


def kernel(points, grid):
    sh = getattr(points, "sharding", None)
    fn = _jit_cache.get(sh)
    if fn is None:
        if sh is not None:
            fmt = Format(Layout(major_to_minor=(0, 1)), sh)
            fn = jax.jit(_run, out_shardings=fmt)
        else:
            fn = jax.jit(_run)
        _jit_cache[sh] = fn
    return fn(points, grid)


# speedup vs baseline: 4.3190x; 1.4727x over previous
import jax
import jax.numpy as jnp
from jax import lax
from jax.experimental import pallas as pl
from jax.experimental.pallas import tpu as pltpu
from jax.experimental.pallas import tpu_sc as plsc
from jax.experimental.layout import Format, Layout

GS = 128
F = 32
N = 2000000
V = GS * GS * GS

NC = 2
NS = 16
NW = NC * NS

C = 640
NIDX = C // 128
NCHUNKS = N // C
NMAXH = (NCHUNKS + 2 * NW - 1) // (2 * NW)

YB = 32
PB = 16000
NPB = N // PB

_RND = 8388608.0


def _tr_body(g_ref, t_ref):
    for yq in range(YB // 4):
        parts = [
            jnp.transpose(g_ref[0, yq * 4 + p], (1, 0)) for p in range(4)
        ]
        t_ref[yq] = jnp.concatenate(parts, axis=1)


def _idx_body(p_ref, o_ref):
    def rnd(t):
        t = jnp.clip(t * (GS - 1.0), 0.0, GS - 1.0)
        return (t + _RND) - _RND

    x = rnd(p_ref[0:1, :])
    y = rnd(p_ref[1:2, :])
    z = rnd(p_ref[2:3, :])
    yq = jnp.floor(y * 0.25)
    yr = y - yq * 4.0
    o_ref[0] = (x * 16384.0 + yq * 512.0 + z * 4.0 + yr).astype(jnp.int32)


def _sc_body(idx_hbm, table_hbm, out_hbm, idx_v, rows_v, si0, si1, sg0, sg1,
             so0, so1):
    wid = lax.axis_index("s") * NC + lax.axis_index("c")
    sem_in = (si0, si1)
    sem_g = (sg0, sg1)
    sem_out = (so0, so1)

    def in_copy(k, s):
        return pltpu.make_async_copy(
            idx_hbm.at[wid + k * NW], idx_v.at[s], sem_in[s]
        )

    def gather_copies(k, s):
        return [
            pltpu.make_async_copy(
                table_hbm.at[idx_v.at[s, j]],
                rows_v.at[s, pl.ds(j * 128, 128)],
                sem_g[s],
            )
            for j in range(NIDX)
        ]

    def out_copy(k, s):
        return pltpu.make_async_copy(
            out_hbm.at[pl.ds((wid + k * NW) * C, C)], rows_v.at[s], sem_out[s]
        )

    def valid(k):
        return wid + k * NW < NCHUNKS

    in_copy(0, 0).start()
    in_copy(1, 1).start()

    def outer(io, carry):
        for b in range(2):
            k = io * 2 + b

            @pl.when(valid(k))
            def _():
                in_copy(k, b).wait()

                @pl.when(io > 0)
                def _():
                    pltpu.make_async_copy(
                        out_hbm.at[pl.ds(0, C)], rows_v.at[b], sem_out[b]
                    ).wait()

                for cp in gather_copies(k, b):
                    cp.start()
                for cp in gather_copies(k, b):
                    cp.wait()

                @pl.when(valid(k + 2))
                def _():
                    in_copy(k + 2, b).start()

                cp = out_copy(k, b)
                cp.start()

        return carry

    lax.fori_loop(0, NMAXH, outer, 0)

    for b in range(2):
        pltpu.make_async_copy(
            out_hbm.at[pl.ds(0, C)], rows_v.at[b], sem_out[b]
        ).wait()


def _run(points, grid):
    g2 = jnp.transpose(grid, (0, 1, 3, 2))
    pts_t = jnp.transpose(points, (1, 0))

    table = pl.pallas_call(
        _tr_body,
        grid=(GS, GS // YB),
        in_specs=[
            pl.BlockSpec((1, YB, F, GS), lambda i, j: (i, j, 0, 0)),
        ],
        out_specs=pl.BlockSpec(
            (YB // 4, GS, 4 * F), lambda i, j: (i * (GS // YB) + j, 0, 0)
        ),
        out_shape=jax.ShapeDtypeStruct((GS * GS // 4, GS, 4 * F), jnp.float32),
    )(g2)
    table = table.reshape(V, F)

    idx = pl.pallas_call(
        _idx_body,
        grid=(NPB,),
        in_specs=[pl.BlockSpec((3, PB), lambda i: (0, i))],
        out_specs=pl.BlockSpec((1, 1, PB), lambda i: (i, 0, 0)),
        out_shape=jax.ShapeDtypeStruct((NPB, 1, PB), jnp.int32),
    )(pts_t)
    idx = idx.reshape(NCHUNKS, NIDX, 128)

    mesh = plsc.VectorSubcoreMesh(core_axis_name="c", subcore_axis_name="s")
    run = pl.kernel(
        _sc_body,
        out_type=jax.ShapeDtypeStruct((N, F), jnp.float32),
        mesh=mesh,
        compiler_params=pltpu.CompilerParams(
            needs_layout_passes=False, use_tc_tiling_on_sc=False
        ),
        scratch_types=[
            pltpu.VMEM((2, NIDX, 128), jnp.int32),
            pltpu.VMEM((2, C, F), jnp.float32),
            pltpu.SemaphoreType.DMA,
            pltpu.SemaphoreType.DMA,
            pltpu.SemaphoreType.DMA,
            pltpu.SemaphoreType.DMA,
            pltpu.SemaphoreType.DMA,
            pltpu.SemaphoreType.DMA,
        ],
    )
    return run(idx, table)


_jit_cache = {}


def kernel(points, grid):
    sh = getattr(points, "sharding", None)
    fn = _jit_cache.get(sh)
    if fn is None:
        if sh is not None:
            fmt = Format(Layout(major_to_minor=(0, 1), tiling=((8,),)), sh)
            fn = jax.jit(_run, out_shardings=fmt)
        else:
            fn = jax.jit(_run)
        _jit_cache[sh] = fn
    return fn(points, grid)
